# Initial kernel scaffold; baseline (speedup 1.0000x reference)
import jax
import jax.numpy as jnp
from jax.experimental import pallas as pl


def kernel(interpolated, rpn_boxes, W1_0, b1_0, W2_0, b2_0, Wo_0, bo_0, W1_1, b1_1, W2_1, b2_1, Wo_1, bo_1, Wf, bf):
    raise NotImplementedError("write your pallas kernel here")


# speedup vs baseline: 1.6155x; 1.6155x over previous
import functools

import jax
import jax.numpy as jnp
from jax.experimental import pallas as pl
from jax.experimental.pallas import tpu as pltpu

N = 5000
NPAD = 5120
RB = 256
D = 129
K = 32
H = 256
TILE = 224.0
THR = 0.5
BIGI = 2 ** 30


def _topk_body(boxes_blk, boxesT, vals_out, idx_out, iou_s, vacc_s, iacc_s):
    rows = boxes_blk[...]
    bT = boxesT[...]
    x1c, y1c, x2c, y2c = bT[0:1, :], bT[1:2, :], bT[2:3, :], bT[3:4, :]
    x1r, y1r = rows[:, 0:1], rows[:, 1:2]
    x2r, y2r = rows[:, 2:3], rows[:, 3:4]
    ltx = jnp.maximum(x1r, x1c)
    lty = jnp.maximum(y1r, y1c)
    rbx = jnp.minimum(x2r, x2c)
    rby = jnp.minimum(y2r, y2c)
    inter = jnp.maximum(rbx - ltx, 0.0) * jnp.maximum(rby - lty, 0.0)
    area_r = (x2r - x1r) * (y2r - y1r)
    area_c = (x2c - x1c) * (y2c - y1c)
    iou = inter / (area_r + area_c - inter + 1e-9)
    iou_s[...] = jnp.where(iou > THR, iou, -1.0)
    vacc_s[...] = jnp.full((RB, K), -1.0, dtype=jnp.float32)
    iacc_s[...] = jnp.zeros((RB, K), dtype=jnp.int32)

    colids = jax.lax.broadcasted_iota(jnp.int32, (RB, NPAD), 1)
    kio = jax.lax.broadcasted_iota(jnp.int32, (RB, K), 1)

    def body(carry):
        k, _ = carry
        cur = iou_s[...]
        m = jnp.max(cur, axis=1, keepdims=True)
        t = jnp.where(cur == m, colids, BIGI)
        a = jnp.min(t, axis=1, keepdims=True)
        vacc_s[...] = jnp.where(kio == k, m, vacc_s[...])
        iacc_s[...] = jnp.where(kio == k, a, iacc_s[...])
        iou_s[...] = jnp.where(colids == a, -1.0, cur)
        return k + 1, jnp.max(m) > 0.0

    def cond(carry):
        k, active = carry
        return jnp.logical_and(k < K, active)

    jax.lax.while_loop(cond, body, (jnp.int32(0), True))
    vals_out[...] = vacc_s[...]
    idx_out[...] = iacc_s[...]


def _topk_pallas(boxes_p, boxesT):
    grid = (NPAD // RB,)
    return pl.pallas_call(
        _topk_body,
        grid=grid,
        in_specs=[
            pl.BlockSpec((RB, 4), lambda i: (i, 0)),
            pl.BlockSpec((8, NPAD), lambda i: (0, 0)),
        ],
        out_specs=[
            pl.BlockSpec((RB, K), lambda i: (i, 0)),
            pl.BlockSpec((RB, K), lambda i: (i, 0)),
        ],
        out_shape=[
            jax.ShapeDtypeStruct((NPAD, K), jnp.float32),
            jax.ShapeDtypeStruct((NPAD, K), jnp.int32),
        ],
        scratch_shapes=[
            pltpu.VMEM((RB, NPAD), jnp.float32),
            pltpu.VMEM((RB, K), jnp.float32),
            pltpu.VMEM((RB, K), jnp.int32),
        ],
    )(boxes_p, boxesT)


def kernel(interpolated, rpn_boxes,
           W1_0, b1_0, W2_0, b2_0, Wo_0, bo_0,
           W1_1, b1_1, W2_1, b2_1, Wo_1, bo_1,
           Wf, bf):
    p = {
        "W1_0": W1_0, "b1_0": b1_0, "W2_0": W2_0, "b2_0": b2_0,
        "Wo_0": Wo_0, "bo_0": bo_0,
        "W1_1": W1_1, "b1_1": b1_1, "W2_1": W2_1, "b2_1": b2_1,
        "Wo_1": Wo_1, "bo_1": bo_1,
    }
    boxes_p = jnp.zeros((NPAD, 4), jnp.float32).at[:N].set(rpn_boxes)
    boxesT = jnp.zeros((8, NPAD), jnp.float32).at[:4].set(boxes_p.T)
    vals, idx = _topk_pallas(boxes_p, boxesT)
    vals = vals[:N]
    idx = idx[:N]

    mask = vals > THR
    nb_boxes = jnp.take(rpn_boxes, idx, axis=0)
    deltas = (nb_boxes - rpn_boxes[:, None, :]) / TILE
    add_info = jnp.concatenate([deltas, vals[..., None]], axis=-1)
    feats = interpolated
    for b in range(2):
        center = jnp.broadcast_to(feats[:, None, :], (N, K, D))
        neighbor = jnp.take(feats, idx, axis=0)
        pairs = jnp.concatenate([center, neighbor, add_info], axis=-1)
        h = jax.nn.relu(pairs @ p[f"W1_{b}"] + p[f"b1_{b}"])
        h = jax.nn.relu(h @ p[f"W2_{b}"] + p[f"b2_{b}"])
        h = jnp.where(mask[..., None], h, -1e30)
        pooled = jnp.max(h, axis=1)
        feats = feats + (pooled @ p[f"Wo_{b}"] + p[f"bo_{b}"])
    return feats @ Wf + bf
